# Initial kernel scaffold; baseline (speedup 1.0000x reference)
import jax
import jax.numpy as jnp
from jax.experimental import pallas as pl


def kernel(x, edge_index, W0, b0, gamma0, beta0, W1, b1, gamma1, beta1, W2, b2, gamma2, beta2):
    raise NotImplementedError("write your pallas kernel here")


# speedup vs baseline: 12.1387x; 12.1387x over previous
import functools

import jax
import jax.numpy as jnp
from jax import lax
from jax.experimental import pallas as pl
from jax.experimental.pallas import tpu as pltpu
from jax.experimental.pallas import tpu_sc as plsc

_N = 10000
_NP = 10240
_D = 128
_CH = 128
_NC = 2
_NT = 16
_NW = _NC * _NT


def _mesh():
    return plsc.VectorSubcoreMesh(core_axis_name="c", subcore_axis_name="s")


@functools.cache
def _make_sc_counts(chunks):
    rows = _NP * 4 // _NT

    @functools.partial(
        pl.kernel,
        out_type=jax.ShapeDtypeStruct((_NC, _NT, rows), jnp.float32),
        mesh=_mesh(),
        scratch_types=[
            pltpu.VMEM((2, _CH), jnp.int32),
            pltpu.VMEM((1, _CH), jnp.int32),
            pltpu.VMEM((1, _CH), jnp.int32),
            pltpu.VMEM((1, _CH), jnp.int32),
            pltpu.VMEM((1, _CH), jnp.float32),
            pltpu.VMEM((1, _CH), jnp.float32),
            pltpu.VMEM_SHARED((_NP * 4,), jnp.float32),
        ],
    )
    def counts_kernel(e_ref, z_ref, o_ref, ib, i0, i1, i2, ones_v, val_v,
                      acc):
        cid = lax.axis_index("c")
        sid = lax.axis_index("s")
        wid = cid * _NT + sid
        pltpu.sync_copy(z_ref.at[sid], acc.at[pl.ds(sid * rows, rows)])
        for k in range(0, _CH, 16):
            ones_v[0, pl.ds(k, 16)] = jnp.full((16,), 1.0, jnp.float32)
        plsc.subcore_barrier()

        @pl.loop(0, chunks)
        def _(j):
            pltpu.sync_copy(e_ref.at[wid, j], ib)
            for k in range(0, _CH, 16):
                s = ib[0, pl.ds(k, 16)]
                d = ib[1, pl.ds(k, 16)]
                i0[0, pl.ds(k, 16)] = s * 4
                i1[0, pl.ds(k, 16)] = d * 4 + 1
                i2[0, pl.ds(k, 16)] = s * 4 + 2
                val_v[0, pl.ds(k, 16)] = jnp.where(
                    s == d, jnp.float32(1.0), jnp.float32(0.0))
            pltpu.sync_copy(ones_v.at[0], acc.at[i0.at[0]], add=True)
            pltpu.sync_copy(ones_v.at[0], acc.at[i1.at[0]], add=True)
            pltpu.sync_copy(val_v.at[0], acc.at[i2.at[0]], add=True)

        plsc.subcore_barrier()
        pltpu.sync_copy(acc.at[pl.ds(sid * rows, rows)], o_ref.at[cid, sid])

    return counts_kernel


@functools.cache
def _make_sc_agg(chunks):
    rows = _NP // _NT

    @functools.partial(
        pl.kernel,
        out_type=jax.ShapeDtypeStruct((_NC, _NP, _D), jnp.float32),
        mesh=_mesh(),
        scratch_types=[
            pltpu.VMEM((2, _CH), jnp.int32),
            pltpu.VMEM((2, _CH), jnp.int32),
            pltpu.VMEM((_CH, _D), jnp.float32),
            pltpu.VMEM((_CH, _D), jnp.float32),
            pltpu.VMEM_SHARED((_NP, _D), jnp.float32),
            pltpu.SemaphoreType.DMA,
            pltpu.SemaphoreType.DMA,
            pltpu.SemaphoreType.DMA,
            pltpu.SemaphoreType.DMA,
        ],
    )
    def agg_kernel(t_ref, e_ref, z_ref, o_ref, ia, ib, ra, rb, acc,
                   sia, sib, sga, sgb):
        cid = lax.axis_index("c")
        sid = lax.axis_index("s")
        wid = cid * _NT + sid
        pltpu.sync_copy(z_ref.at[pl.ds(sid * rows, rows)],
                        acc.at[pl.ds(sid * rows, rows)])
        plsc.subcore_barrier()

        pltpu.async_copy(e_ref.at[wid, 0], ia, sia)
        pltpu.async_copy(e_ref.at[wid, 1], ib, sib)
        pltpu.make_async_copy(e_ref.at[wid, 0], ia, sia).wait()
        pltpu.async_copy(t_ref.at[ia.at[0]], ra, sga)

        @pl.loop(0, chunks, step=2)
        def _(j):
            pltpu.make_async_copy(e_ref.at[wid, j + 1], ib, sib).wait()
            pltpu.make_async_copy(t_ref.at[ia.at[0]], ra, sga).wait()
            pltpu.async_copy(t_ref.at[ib.at[0]], rb, sgb)
            pltpu.sync_copy(ra, acc.at[ia.at[1]], add=True)

            @pl.when(j + 2 < chunks)
            def _():
                pltpu.async_copy(e_ref.at[wid, j + 2], ia, sia)

            pltpu.make_async_copy(t_ref.at[ib.at[0]], rb, sgb).wait()
            pltpu.sync_copy(rb, acc.at[ib.at[1]], add=True)

            @pl.when(j + 2 < chunks)
            def _():
                pltpu.make_async_copy(e_ref.at[wid, j + 2], ia, sia).wait()
                pltpu.async_copy(t_ref.at[ia.at[0]], ra, sga)

            @pl.when(j + 3 < chunks)
            def _():
                pltpu.async_copy(e_ref.at[wid, j + 3], ib, sib)

        plsc.subcore_barrier()
        pltpu.sync_copy(acc.at[pl.ds(sid * rows, rows)],
                        o_ref.at[cid, pl.ds(sid * rows, rows)])

    return agg_kernel


def _tc_matmul(x, w):
    def body(x_ref, w_ref, o_ref):
        o_ref[...] = jnp.dot(x_ref[...], w_ref[...],
                             preferred_element_type=jnp.float32)

    return pl.pallas_call(
        body,
        out_shape=jax.ShapeDtypeStruct((x.shape[0], w.shape[1]), jnp.float32),
    )(x, w)


def _tc_scale0(u, counts):
    def body(u_ref, c_ref, o_ref):
        cnt = c_ref[0] + c_ref[1]
        deg = jnp.maximum(cnt[:, 0:1] - cnt[:, 2:3] + 1.0, 1.0)
        nsrc = lax.rsqrt(deg)
        o_ref[0:_N, :] = nsrc[0:_N] * u_ref[...]
        o_ref[_N:_NP, :] = jnp.zeros((_NP - _N, _D), jnp.float32)

    return pl.pallas_call(
        body, out_shape=jax.ShapeDtypeStruct((_NP, _D), jnp.float32),
    )(u, counts)


def _tc_epilogue(acc, hws, counts, b, g, be, wn):
    has_next = wn is not None
    outs = [jax.ShapeDtypeStruct((_N, _D), jnp.float32)]
    if has_next:
        outs.append(jax.ShapeDtypeStruct((_NP, _D), jnp.float32))

    def body(acc_ref, hws_ref, c_ref, b_ref, g_ref, be_ref, *rest):
        if has_next:
            wn_ref, h_ref, hn_ref = rest
        else:
            (h_ref,) = rest
        cnt = c_ref[0] + c_ref[1]
        c = cnt[0:_N, 2:3]
        ndst = lax.rsqrt(jnp.maximum(cnt[0:_N, 1:2] - c + 1.0, 1.0))
        agg = (acc_ref[0, 0:_N, :] + acc_ref[1, 0:_N, :]
               + (1.0 - c) * hws_ref[0:_N, :])
        pre = ndst * agg + b_ref[...]
        m = jnp.mean(pre, axis=0)
        msq = jnp.mean(pre * pre, axis=0)
        var = msq - m * m
        h = jnp.maximum(
            (pre - m) * lax.rsqrt(var + 1e-5) * g_ref[...] + be_ref[...], 0.0)
        h_ref[...] = h
        if has_next:
            nsrc = lax.rsqrt(jnp.maximum(cnt[0:_N, 0:1] - c + 1.0, 1.0))
            u = jnp.dot(h, wn_ref[...], preferred_element_type=jnp.float32)
            hn_ref[0:_N, :] = nsrc * u
            hn_ref[_N:_NP, :] = jnp.zeros((_NP - _N, _D), jnp.float32)

    args = [acc, hws, counts, b, g, be] + ([wn] if has_next else [])
    res = pl.pallas_call(body, out_shape=outs)(*args)
    return tuple(res)


def kernel(x, edge_index, W0, b0, gamma0, beta0, W1, b1, gamma1, beta1,
           W2, b2, gamma2, beta2):
    e = edge_index.astype(jnp.int32)
    E = e.shape[1]
    block = _NW * _CH * 2
    epad = -(-E // block) * block
    chunks = epad // (_NW * _CH)
    n_pad = epad - E
    dump = _NP - _N
    pidx = jnp.arange(n_pad, dtype=jnp.int32)
    psrc = _N + pidx % dump
    pdst = _N + (pidx * 7 + 13) % dump
    src = jnp.concatenate([e[0], psrc])
    dst = jnp.concatenate([e[1], pdst])
    e_all = jnp.stack([src, dst]).reshape(2, _NW, chunks, _CH)
    e_all = jnp.transpose(e_all, (1, 2, 0, 3))
    zc = jnp.zeros((_NT, _NP * 4 // _NT), jnp.float32)
    za = jnp.zeros((_NP, _D), jnp.float32)

    counts = _make_sc_counts(chunks)(e_all, zc)
    counts = counts.reshape(_NC, _NP, 4)
    u0 = _tc_matmul(x, W0)
    hws0 = _tc_scale0(u0, counts)

    acc0 = _make_sc_agg(chunks)(hws0, e_all, za)
    h1, hws1 = _tc_epilogue(acc0, hws0, counts, b0, gamma0, beta0, W1)
    acc1 = _make_sc_agg(chunks)(hws1, e_all, za)
    h2, hws2 = _tc_epilogue(acc1, hws1, counts, b1, gamma1, beta1, W2)
    acc2 = _make_sc_agg(chunks)(hws2, e_all, za)
    (h3,) = _tc_epilogue(acc2, hws2, counts, b2, gamma2, beta2, None)

    return (x, h1, h2, h3)


# speedup vs baseline: 15.7522x; 1.2977x over previous
import functools

import jax
import jax.numpy as jnp
from jax import lax
from jax.experimental import pallas as pl
from jax.experimental.pallas import tpu as pltpu
from jax.experimental.pallas import tpu_sc as plsc

_N = 10000
_NP = 10240
_D = 128
_CH = 64
_BK = 4
_NC = 2
_NT = 16
_NW = _NC * _NT


def _mesh():
    return plsc.VectorSubcoreMesh(core_axis_name="c", subcore_axis_name="s")


@functools.cache
def _make_sc_counts(nblocks):
    rows = _NP * 4 // _NT

    @functools.partial(
        pl.kernel,
        out_type=jax.ShapeDtypeStruct((_NC, _NT, rows), jnp.float32),
        mesh=_mesh(),
        scratch_types=[
            pltpu.VMEM((_BK, 2, _CH), jnp.int32),
            pltpu.VMEM((_BK, 2, _CH), jnp.int32),
            pltpu.VMEM((3 * _BK, _CH), jnp.int32),
            pltpu.VMEM((_BK, _CH), jnp.float32),
            pltpu.VMEM((1, _CH), jnp.float32),
            pltpu.VMEM_SHARED((_NP * 4,), jnp.float32),
            pltpu.SemaphoreType.DMA,
            pltpu.SemaphoreType.DMA,
            pltpu.SemaphoreType.DMA,
        ],
    )
    def counts_kernel(e_ref, z_ref, o_ref, bA, bB, ix, vv, ones_v, acc,
                      sbA, sbB, ssc):
        cid = lax.axis_index("c")
        sid = lax.axis_index("s")
        wid = cid * _NT + sid
        pltpu.sync_copy(z_ref.at[sid], acc.at[pl.ds(sid * rows, rows)])
        for t in range(0, _CH, 16):
            ones_v[0, pl.ds(t, 16)] = jnp.full((16,), 1.0, jnp.float32)
        plsc.subcore_barrier()

        def compute(blk):
            for k in range(_BK):
                for t in range(0, _CH, 16):
                    s = blk[k, 0, pl.ds(t, 16)]
                    d = blk[k, 1, pl.ds(t, 16)]
                    ix[3 * k + 0, pl.ds(t, 16)] = s * 4
                    ix[3 * k + 1, pl.ds(t, 16)] = d * 4 + 1
                    ix[3 * k + 2, pl.ds(t, 16)] = s * 4 + 2
                    vv[k, pl.ds(t, 16)] = jnp.where(
                        s == d, jnp.float32(1.0), jnp.float32(0.0))

        def fire(k):
            pltpu.async_copy(ones_v.at[0], acc.at[ix.at[3 * k + 0]], ssc,
                             add=True)
            pltpu.async_copy(ones_v.at[0], acc.at[ix.at[3 * k + 1]], ssc,
                             add=True)
            pltpu.async_copy(vv.at[k], acc.at[ix.at[3 * k + 2]], ssc,
                             add=True)

        def drain():
            for k in range(_BK):
                pltpu.make_async_copy(
                    ones_v.at[0], acc.at[ix.at[3 * k + 0]], ssc).wait()
                pltpu.make_async_copy(
                    ones_v.at[0], acc.at[ix.at[3 * k + 1]], ssc).wait()
                pltpu.make_async_copy(
                    vv.at[k], acc.at[ix.at[3 * k + 2]], ssc).wait()

        pltpu.async_copy(e_ref.at[wid, 0], bA, sbA)
        pltpu.async_copy(e_ref.at[wid, 1], bB, sbB)

        @pl.loop(0, nblocks, step=2)
        def _(b):
            pltpu.make_async_copy(e_ref.at[wid, b], bA, sbA).wait()

            @pl.when(b >= 2)
            def _():
                drain()

            compute(bA)
            for k in range(_BK):
                fire(k)

            @pl.when(b + 2 < nblocks)
            def _():
                pltpu.async_copy(e_ref.at[wid, b + 2], bA, sbA)

            pltpu.make_async_copy(e_ref.at[wid, b + 1], bB, sbB).wait()
            drain()
            compute(bB)
            for k in range(_BK):
                fire(k)

            @pl.when(b + 3 < nblocks)
            def _():
                pltpu.async_copy(e_ref.at[wid, b + 3], bB, sbB)

        drain()
        plsc.subcore_barrier()
        pltpu.sync_copy(acc.at[pl.ds(sid * rows, rows)], o_ref.at[cid, sid])

    return counts_kernel


@functools.cache
def _make_sc_agg(nblocks):
    rows = _NP // _NT

    @functools.partial(
        pl.kernel,
        out_type=jax.ShapeDtypeStruct((_NC, _NP, _D), jnp.float32),
        mesh=_mesh(),
        scratch_types=[
            pltpu.VMEM((_BK, 2, _CH), jnp.int32),
            pltpu.VMEM((_BK, 2, _CH), jnp.int32),
            pltpu.VMEM((_CH, _D), jnp.float32),
            pltpu.VMEM((_CH, _D), jnp.float32),
            pltpu.VMEM((_CH, _D), jnp.float32),
            pltpu.VMEM((_CH, _D), jnp.float32),
            pltpu.VMEM_SHARED((_NP, _D), jnp.float32),
            pltpu.SemaphoreType.DMA,
            pltpu.SemaphoreType.DMA,
            pltpu.SemaphoreType.DMA,
            pltpu.SemaphoreType.DMA,
            pltpu.SemaphoreType.DMA,
            pltpu.SemaphoreType.DMA,
            pltpu.SemaphoreType.DMA,
            pltpu.SemaphoreType.DMA,
            pltpu.SemaphoreType.DMA,
            pltpu.SemaphoreType.DMA,
        ],
    )
    def agg_kernel(t_ref, e_ref, z_ref, o_ref, iA, iB, r0, r1, r2, r3, acc,
                   siA, siB, sg0, sg1, sg2, sg3, ss0, ss1, ss2, ss3):
        r = (r0, r1, r2, r3)
        sg = (sg0, sg1, sg2, sg3)
        ss = (ss0, ss1, ss2, ss3)
        cid = lax.axis_index("c")
        sid = lax.axis_index("s")
        wid = cid * _NT + sid
        pltpu.sync_copy(z_ref.at[pl.ds(sid * rows, rows)],
                        acc.at[pl.ds(sid * rows, rows)])
        plsc.subcore_barrier()

        pltpu.async_copy(e_ref.at[wid, 0], iA, siA)
        pltpu.async_copy(e_ref.at[wid, 1], iB, siB)
        pltpu.make_async_copy(e_ref.at[wid, 0], iA, siA).wait()
        for k in range(_BK):
            pltpu.async_copy(t_ref.at[iA.at[k, 0]], r[k], sg[k])

        @pl.loop(0, nblocks, step=2)
        def _(b):
            for k in range(_BK):
                pltpu.make_async_copy(t_ref.at[iA.at[k, 0]], r[k],
                                      sg[k]).wait()
                pltpu.async_copy(r[k], acc.at[iA.at[k, 1]], ss[k], add=True)
            pltpu.make_async_copy(e_ref.at[wid, b + 1], iB, siB).wait()
            for k in range(_BK):
                pltpu.make_async_copy(r[k], acc.at[iA.at[k, 1]],
                                      ss[k]).wait()
                pltpu.async_copy(t_ref.at[iB.at[k, 0]], r[k], sg[k])
            @pl.when(b + 2 < nblocks)
            def _():
                pltpu.async_copy(e_ref.at[wid, b + 2], iA, siA)

            for k in range(_BK):
                pltpu.make_async_copy(t_ref.at[iB.at[k, 0]], r[k],
                                      sg[k]).wait()
                pltpu.async_copy(r[k], acc.at[iB.at[k, 1]], ss[k], add=True)
            @pl.when(b + 2 < nblocks)
            def _():
                pltpu.make_async_copy(e_ref.at[wid, b + 2], iA, siA).wait()
            for k in range(_BK):
                pltpu.make_async_copy(r[k], acc.at[iB.at[k, 1]],
                                      ss[k]).wait()

                @pl.when(b + 2 < nblocks)
                def _():
                    pltpu.async_copy(t_ref.at[iA.at[k, 0]], r[k], sg[k])

            @pl.when(b + 3 < nblocks)
            def _():
                pltpu.async_copy(e_ref.at[wid, b + 3], iB, siB)

        plsc.subcore_barrier()
        pltpu.sync_copy(acc.at[pl.ds(sid * rows, rows)],
                        o_ref.at[cid, pl.ds(sid * rows, rows)])

    return agg_kernel


def _tc_scale0(x, counts, w):
    def body(x_ref, c_ref, w_ref, o_ref):
        cnt = c_ref[0] + c_ref[1]
        deg = jnp.maximum(cnt[:, 0:1] - cnt[:, 2:3] + 1.0, 1.0)
        nsrc = lax.rsqrt(deg)
        u = jnp.dot(nsrc[0:_N] * x_ref[...], w_ref[...],
                    preferred_element_type=jnp.float32)
        o_ref[0:_N, :] = u
        o_ref[_N:_NP, :] = jnp.zeros((_NP - _N, _D), jnp.float32)

    return pl.pallas_call(
        body, out_shape=jax.ShapeDtypeStruct((_NP, _D), jnp.float32),
    )(x, counts, w)


def _tc_epilogue(acc, hws, counts, b, g, be, wn):
    has_next = wn is not None
    outs = [jax.ShapeDtypeStruct((_N, _D), jnp.float32)]
    if has_next:
        outs.append(jax.ShapeDtypeStruct((_NP, _D), jnp.float32))

    def body(acc_ref, hws_ref, c_ref, b_ref, g_ref, be_ref, *rest):
        if has_next:
            wn_ref, h_ref, hn_ref = rest
        else:
            (h_ref,) = rest
        cnt = c_ref[0] + c_ref[1]
        c = cnt[0:_N, 2:3]
        ndst = lax.rsqrt(jnp.maximum(cnt[0:_N, 1:2] - c + 1.0, 1.0))
        agg = (acc_ref[0, 0:_N, :] + acc_ref[1, 0:_N, :]
               + (1.0 - c) * hws_ref[0:_N, :])
        pre = ndst * agg + b_ref[...]
        m = jnp.mean(pre, axis=0)
        msq = jnp.mean(pre * pre, axis=0)
        var = msq - m * m
        h = jnp.maximum(
            (pre - m) * lax.rsqrt(var + 1e-5) * g_ref[...] + be_ref[...], 0.0)
        h_ref[...] = h
        if has_next:
            nsrc = lax.rsqrt(jnp.maximum(cnt[0:_N, 0:1] - c + 1.0, 1.0))
            u = jnp.dot(nsrc * h, wn_ref[...],
                        preferred_element_type=jnp.float32)
            hn_ref[0:_N, :] = u
            hn_ref[_N:_NP, :] = jnp.zeros((_NP - _N, _D), jnp.float32)

    args = [acc, hws, counts, b, g, be] + ([wn] if has_next else [])
    res = pl.pallas_call(body, out_shape=outs)(*args)
    return tuple(res)


def kernel(x, edge_index, W0, b0, gamma0, beta0, W1, b1, gamma1, beta1,
           W2, b2, gamma2, beta2):
    e = edge_index.astype(jnp.int32)
    E = e.shape[1]
    block = _NW * _BK * _CH * 2
    epad = -(-E // block) * block
    nblocks = epad // (_NW * _BK * _CH)
    n_pad = epad - E
    dump = _NP - _N
    pidx = jnp.arange(n_pad, dtype=jnp.int32)
    psrc = _N + pidx % dump
    pdst = _N + (pidx * 7 + 13) % dump
    src = jnp.concatenate([e[0], psrc])
    dst = jnp.concatenate([e[1], pdst])
    e_all = jnp.stack([src, dst]).reshape(2, _NW, nblocks, _BK, _CH)
    e_all = jnp.transpose(e_all, (1, 2, 3, 0, 4))
    zc = jnp.zeros((_NT, _NP * 4 // _NT), jnp.float32)
    za = jnp.zeros((_NP, _D), jnp.float32)

    counts = _make_sc_counts(nblocks)(e_all, zc)
    counts = counts.reshape(_NC, _NP, 4)
    hws0 = _tc_scale0(x, counts, W0)

    acc0 = _make_sc_agg(nblocks)(hws0, e_all, za)
    h1, hws1 = _tc_epilogue(acc0, hws0, counts, b0, gamma0, beta0, W1)
    acc1 = _make_sc_agg(nblocks)(hws1, e_all, za)
    h2, hws2 = _tc_epilogue(acc1, hws1, counts, b1, gamma1, beta1, W2)
    acc2 = _make_sc_agg(nblocks)(hws2, e_all, za)
    (h3,) = _tc_epilogue(acc2, hws2, counts, b2, gamma2, beta2, None)

    return (x, h1, h2, h3)


# speedup vs baseline: 16.0241x; 1.0173x over previous
import functools

import jax
import jax.numpy as jnp
from jax import lax
from jax.experimental import pallas as pl
from jax.experimental.pallas import tpu as pltpu
from jax.experimental.pallas import tpu_sc as plsc

_N = 10000
_NP = 10240
_D = 128
_CH = 64
_BK = 4
_NC = 2
_NT = 16
_NW = _NC * _NT


def _mesh():
    return plsc.VectorSubcoreMesh(core_axis_name="c", subcore_axis_name="s")


@functools.cache
def _make_sc_counts(nblocks):
    rows = _NP * 4 // _NT

    @functools.partial(
        pl.kernel,
        out_type=jax.ShapeDtypeStruct((_NC, _NT, rows), jnp.float32),
        mesh=_mesh(),
        scratch_types=[
            pltpu.VMEM((_BK, 2, _CH), jnp.int32),
            pltpu.VMEM((_BK, 2, _CH), jnp.int32),
            pltpu.VMEM((3 * _BK, _CH), jnp.int32),
            pltpu.VMEM((_BK, _CH), jnp.float32),
            pltpu.VMEM((1, _CH), jnp.float32),
            pltpu.VMEM_SHARED((_NP * 4,), jnp.float32),
            pltpu.SemaphoreType.DMA,
            pltpu.SemaphoreType.DMA,
            pltpu.SemaphoreType.DMA,
        ],
    )
    def counts_kernel(e_ref, z_ref, o_ref, bA, bB, ix, vv, ones_v, acc,
                      sbA, sbB, ssc):
        cid = lax.axis_index("c")
        sid = lax.axis_index("s")
        wid = cid * _NT + sid
        pltpu.sync_copy(z_ref.at[sid], acc.at[pl.ds(sid * rows, rows)])
        for t in range(0, _CH, 16):
            ones_v[0, pl.ds(t, 16)] = jnp.full((16,), 1.0, jnp.float32)
        plsc.subcore_barrier()

        def compute(blk):
            for k in range(_BK):
                for t in range(0, _CH, 16):
                    s = blk[k, 0, pl.ds(t, 16)]
                    d = blk[k, 1, pl.ds(t, 16)]
                    ix[3 * k + 0, pl.ds(t, 16)] = s * 4
                    ix[3 * k + 1, pl.ds(t, 16)] = d * 4 + 1
                    ix[3 * k + 2, pl.ds(t, 16)] = s * 4 + 2
                    vv[k, pl.ds(t, 16)] = jnp.where(
                        s == d, jnp.float32(1.0), jnp.float32(0.0))

        def fire(k):
            pltpu.async_copy(ones_v.at[0], acc.at[ix.at[3 * k + 0]], ssc,
                             add=True)
            pltpu.async_copy(ones_v.at[0], acc.at[ix.at[3 * k + 1]], ssc,
                             add=True)
            pltpu.async_copy(vv.at[k], acc.at[ix.at[3 * k + 2]], ssc,
                             add=True)

        def drain():
            for k in range(_BK):
                pltpu.make_async_copy(
                    ones_v.at[0], acc.at[ix.at[3 * k + 0]], ssc).wait()
                pltpu.make_async_copy(
                    ones_v.at[0], acc.at[ix.at[3 * k + 1]], ssc).wait()
                pltpu.make_async_copy(
                    vv.at[k], acc.at[ix.at[3 * k + 2]], ssc).wait()

        pltpu.async_copy(e_ref.at[wid, 0], bA, sbA)
        pltpu.async_copy(e_ref.at[wid, 1], bB, sbB)

        @pl.loop(0, nblocks, step=2)
        def _(b):
            pltpu.make_async_copy(e_ref.at[wid, b], bA, sbA).wait()

            @pl.when(b >= 2)
            def _():
                drain()

            compute(bA)
            for k in range(_BK):
                fire(k)

            @pl.when(b + 2 < nblocks)
            def _():
                pltpu.async_copy(e_ref.at[wid, b + 2], bA, sbA)

            pltpu.make_async_copy(e_ref.at[wid, b + 1], bB, sbB).wait()
            drain()
            compute(bB)
            for k in range(_BK):
                fire(k)

            @pl.when(b + 3 < nblocks)
            def _():
                pltpu.async_copy(e_ref.at[wid, b + 3], bB, sbB)

        drain()
        plsc.subcore_barrier()
        pltpu.sync_copy(acc.at[pl.ds(sid * rows, rows)], o_ref.at[cid, sid])

    return counts_kernel


@functools.cache
def _make_sc_agg(nblocks):
    rows = _NP // _NT

    @functools.partial(
        pl.kernel,
        out_type=jax.ShapeDtypeStruct((_NC, _NP, _D), jnp.float32),
        mesh=_mesh(),
        scratch_types=[
            pltpu.VMEM((_BK, 2, _CH), jnp.int32),
            pltpu.VMEM((_BK, 2, _CH), jnp.int32),
            pltpu.VMEM((_CH, _D), jnp.float32),
            pltpu.VMEM((_CH, _D), jnp.float32),
            pltpu.VMEM((_CH, _D), jnp.float32),
            pltpu.VMEM((_CH, _D), jnp.float32),
            pltpu.VMEM_SHARED((_NP, _D), jnp.float32),
            pltpu.SemaphoreType.DMA,
            pltpu.SemaphoreType.DMA,
            pltpu.SemaphoreType.DMA,
            pltpu.SemaphoreType.DMA,
            pltpu.SemaphoreType.DMA,
            pltpu.SemaphoreType.DMA,
            pltpu.SemaphoreType.DMA,
            pltpu.SemaphoreType.DMA,
            pltpu.SemaphoreType.DMA,
            pltpu.SemaphoreType.DMA,
            pltpu.SemaphoreType.DMA,
        ],
    )
    def agg_kernel(t_ref, e_ref, z_ref, o_ref, iA, iB, r0, r1, r2, r3, acc,
                   siA, siB, sg0, sg1, sg2, sg3, ss0, ss1, ss2, ss3, sz):
        r = (r0, r1, r2, r3)
        sg = (sg0, sg1, sg2, sg3)
        ss = (ss0, ss1, ss2, ss3)
        cid = lax.axis_index("c")
        sid = lax.axis_index("s")
        wid = cid * _NT + sid
        zcopy = pltpu.async_copy(z_ref.at[pl.ds(sid * rows, rows)],
                                 acc.at[pl.ds(sid * rows, rows)], sz)
        pltpu.async_copy(e_ref.at[wid, 0], iA, siA)
        pltpu.async_copy(e_ref.at[wid, 1], iB, siB)
        pltpu.make_async_copy(e_ref.at[wid, 0], iA, siA).wait()
        for k in range(_BK):
            pltpu.async_copy(t_ref.at[iA.at[k, 0]], r[k], sg[k])
        zcopy.wait()
        plsc.subcore_barrier()

        @pl.loop(0, nblocks, step=2)
        def _(b):
            for k in range(_BK):
                pltpu.make_async_copy(t_ref.at[iA.at[k, 0]], r[k],
                                      sg[k]).wait()
                pltpu.async_copy(r[k], acc.at[iA.at[k, 1]], ss[k], add=True)
            pltpu.make_async_copy(e_ref.at[wid, b + 1], iB, siB).wait()
            for k in range(_BK):
                pltpu.make_async_copy(r[k], acc.at[iA.at[k, 1]],
                                      ss[k]).wait()
                pltpu.async_copy(t_ref.at[iB.at[k, 0]], r[k], sg[k])
            @pl.when(b + 2 < nblocks)
            def _():
                pltpu.async_copy(e_ref.at[wid, b + 2], iA, siA)

            for k in range(_BK):
                pltpu.make_async_copy(t_ref.at[iB.at[k, 0]], r[k],
                                      sg[k]).wait()
                pltpu.async_copy(r[k], acc.at[iB.at[k, 1]], ss[k], add=True)
            @pl.when(b + 2 < nblocks)
            def _():
                pltpu.make_async_copy(e_ref.at[wid, b + 2], iA, siA).wait()
            for k in range(_BK):
                pltpu.make_async_copy(r[k], acc.at[iB.at[k, 1]],
                                      ss[k]).wait()

                @pl.when(b + 2 < nblocks)
                def _():
                    pltpu.async_copy(t_ref.at[iA.at[k, 0]], r[k], sg[k])

            @pl.when(b + 3 < nblocks)
            def _():
                pltpu.async_copy(e_ref.at[wid, b + 3], iB, siB)

        plsc.subcore_barrier()
        pltpu.sync_copy(acc.at[pl.ds(sid * rows, rows)],
                        o_ref.at[cid, pl.ds(sid * rows, rows)])

    return agg_kernel


def _tc_scale0(x, counts, w):
    def body(x_ref, c_ref, w_ref, o_ref):
        cnt = c_ref[0] + c_ref[1]
        deg = jnp.maximum(cnt[:, 0:1] - cnt[:, 2:3] + 1.0, 1.0)
        nsrc = lax.rsqrt(deg)
        u = jnp.dot(nsrc[0:_N] * x_ref[...], w_ref[...],
                    preferred_element_type=jnp.float32)
        o_ref[0:_N, :] = u
        o_ref[_N:_NP, :] = jnp.zeros((_NP - _N, _D), jnp.float32)

    return pl.pallas_call(
        body, out_shape=jax.ShapeDtypeStruct((_NP, _D), jnp.float32),
    )(x, counts, w)


def _tc_epilogue(acc, hws, counts, b, g, be, wn):
    has_next = wn is not None
    outs = [jax.ShapeDtypeStruct((_N, _D), jnp.float32)]
    if has_next:
        outs.append(jax.ShapeDtypeStruct((_NP, _D), jnp.float32))

    def body(acc_ref, hws_ref, c_ref, b_ref, g_ref, be_ref, *rest):
        if has_next:
            wn_ref, h_ref, hn_ref = rest
        else:
            (h_ref,) = rest
        cnt = c_ref[0] + c_ref[1]
        c = cnt[0:_N, 2:3]
        ndst = lax.rsqrt(jnp.maximum(cnt[0:_N, 1:2] - c + 1.0, 1.0))
        agg = (acc_ref[0, 0:_N, :] + acc_ref[1, 0:_N, :]
               + (1.0 - c) * hws_ref[0:_N, :])
        pre = ndst * agg + b_ref[...]
        m = jnp.mean(pre, axis=0)
        msq = jnp.mean(pre * pre, axis=0)
        var = msq - m * m
        h = jnp.maximum(
            (pre - m) * lax.rsqrt(var + 1e-5) * g_ref[...] + be_ref[...], 0.0)
        h_ref[...] = h
        if has_next:
            nsrc = lax.rsqrt(jnp.maximum(cnt[0:_N, 0:1] - c + 1.0, 1.0))
            u = jnp.dot(nsrc * h, wn_ref[...],
                        preferred_element_type=jnp.float32)
            hn_ref[0:_N, :] = u
            hn_ref[_N:_NP, :] = jnp.zeros((_NP - _N, _D), jnp.float32)

    args = [acc, hws, counts, b, g, be] + ([wn] if has_next else [])
    res = pl.pallas_call(body, out_shape=outs)(*args)
    return tuple(res)


def kernel(x, edge_index, W0, b0, gamma0, beta0, W1, b1, gamma1, beta1,
           W2, b2, gamma2, beta2):
    e = edge_index.astype(jnp.int32)
    E = e.shape[1]
    block = _NW * _BK * _CH * 2
    epad = -(-E // block) * block
    nblocks = epad // (_NW * _BK * _CH)
    n_pad = epad - E
    dump = _NP - _N
    pidx = jnp.arange(n_pad, dtype=jnp.int32)
    psrc = _N + pidx % dump
    pdst = _N + (pidx * 7 + 13) % dump
    src = jnp.concatenate([e[0], psrc])
    dst = jnp.concatenate([e[1], pdst])
    e_all = jnp.stack([src, dst]).reshape(2, _NW, nblocks, _BK, _CH)
    e_all = jnp.transpose(e_all, (1, 2, 3, 0, 4))
    zc = jnp.zeros((_NT, _NP * 4 // _NT), jnp.float32)
    za = jnp.zeros((_NP, _D), jnp.float32)

    counts = _make_sc_counts(nblocks)(e_all, zc)
    counts = counts.reshape(_NC, _NP, 4)
    hws0 = _tc_scale0(x, counts, W0)

    acc0 = _make_sc_agg(nblocks)(hws0, e_all, za)
    h1, hws1 = _tc_epilogue(acc0, hws0, counts, b0, gamma0, beta0, W1)
    acc1 = _make_sc_agg(nblocks)(hws1, e_all, za)
    h2, hws2 = _tc_epilogue(acc1, hws1, counts, b1, gamma1, beta1, W2)
    acc2 = _make_sc_agg(nblocks)(hws2, e_all, za)
    (h3,) = _tc_epilogue(acc2, hws2, counts, b2, gamma2, beta2, None)

    return (x, h1, h2, h3)


# speedup vs baseline: 16.2947x; 1.0169x over previous
import functools

import jax
import jax.numpy as jnp
from jax import lax
from jax.experimental import pallas as pl
from jax.experimental.pallas import tpu as pltpu
from jax.experimental.pallas import tpu_sc as plsc

_N = 10000
_NP = 10240
_D = 128
_CH = 80
_BK = 4
_NC = 2
_NT = 16
_NW = _NC * _NT


def _mesh():
    return plsc.VectorSubcoreMesh(core_axis_name="c", subcore_axis_name="s")


@functools.cache
def _make_sc_counts(nblocks):
    rows = _NP * 4 // _NT

    @functools.partial(
        pl.kernel,
        out_type=jax.ShapeDtypeStruct((_NC, _NT, rows), jnp.float32),
        mesh=_mesh(),
        scratch_types=[
            pltpu.VMEM((_BK, 2, _CH), jnp.int32),
            pltpu.VMEM((_BK, 2, _CH), jnp.int32),
            pltpu.VMEM((3 * _BK, _CH), jnp.int32),
            pltpu.VMEM((_BK, _CH), jnp.float32),
            pltpu.VMEM((1, _CH), jnp.float32),
            pltpu.VMEM_SHARED((_NP * 4,), jnp.float32),
            pltpu.SemaphoreType.DMA,
            pltpu.SemaphoreType.DMA,
            pltpu.SemaphoreType.DMA,
        ],
    )
    def counts_kernel(e_ref, z_ref, o_ref, bA, bB, ix, vv, ones_v, acc,
                      sbA, sbB, ssc):
        cid = lax.axis_index("c")
        sid = lax.axis_index("s")
        wid = cid * _NT + sid
        pltpu.sync_copy(z_ref.at[sid], acc.at[pl.ds(sid * rows, rows)])
        for t in range(0, _CH, 16):
            ones_v[0, pl.ds(t, 16)] = jnp.full((16,), 1.0, jnp.float32)
        plsc.subcore_barrier()

        def compute(blk):
            for k in range(_BK):
                for t in range(0, _CH, 16):
                    s = blk[k, 0, pl.ds(t, 16)]
                    d = blk[k, 1, pl.ds(t, 16)]
                    ix[3 * k + 0, pl.ds(t, 16)] = s * 4
                    ix[3 * k + 1, pl.ds(t, 16)] = d * 4 + 1
                    ix[3 * k + 2, pl.ds(t, 16)] = s * 4 + 2
                    vv[k, pl.ds(t, 16)] = jnp.where(
                        s == d, jnp.float32(1.0), jnp.float32(0.0))

        def fire(k):
            pltpu.async_copy(ones_v.at[0], acc.at[ix.at[3 * k + 0]], ssc,
                             add=True)
            pltpu.async_copy(ones_v.at[0], acc.at[ix.at[3 * k + 1]], ssc,
                             add=True)
            pltpu.async_copy(vv.at[k], acc.at[ix.at[3 * k + 2]], ssc,
                             add=True)

        def drain():
            for k in range(_BK):
                pltpu.make_async_copy(
                    ones_v.at[0], acc.at[ix.at[3 * k + 0]], ssc).wait()
                pltpu.make_async_copy(
                    ones_v.at[0], acc.at[ix.at[3 * k + 1]], ssc).wait()
                pltpu.make_async_copy(
                    vv.at[k], acc.at[ix.at[3 * k + 2]], ssc).wait()

        pltpu.async_copy(e_ref.at[wid, 0], bA, sbA)
        pltpu.async_copy(e_ref.at[wid, 1], bB, sbB)

        @pl.loop(0, nblocks, step=2)
        def _(b):
            pltpu.make_async_copy(e_ref.at[wid, b], bA, sbA).wait()

            @pl.when(b >= 2)
            def _():
                drain()

            compute(bA)
            for k in range(_BK):
                fire(k)

            @pl.when(b + 2 < nblocks)
            def _():
                pltpu.async_copy(e_ref.at[wid, b + 2], bA, sbA)

            pltpu.make_async_copy(e_ref.at[wid, b + 1], bB, sbB).wait()
            drain()
            compute(bB)
            for k in range(_BK):
                fire(k)

            @pl.when(b + 3 < nblocks)
            def _():
                pltpu.async_copy(e_ref.at[wid, b + 3], bB, sbB)

        drain()
        plsc.subcore_barrier()
        pltpu.sync_copy(acc.at[pl.ds(sid * rows, rows)], o_ref.at[cid, sid])

    return counts_kernel


@functools.cache
def _make_sc_agg(nblocks):
    rows = _NP // _NT

    @functools.partial(
        pl.kernel,
        out_type=jax.ShapeDtypeStruct((_NC, _NP, _D), jnp.float32),
        mesh=_mesh(),
        scratch_types=[
            pltpu.VMEM((_BK, 2, _CH), jnp.int32),
            pltpu.VMEM((_BK, 2, _CH), jnp.int32),
            pltpu.VMEM((_CH, _D), jnp.float32),
            pltpu.VMEM((_CH, _D), jnp.float32),
            pltpu.VMEM((_CH, _D), jnp.float32),
            pltpu.VMEM((_CH, _D), jnp.float32),
            pltpu.VMEM_SHARED((_NP, _D), jnp.float32),
            pltpu.SemaphoreType.DMA,
            pltpu.SemaphoreType.DMA,
            pltpu.SemaphoreType.DMA,
            pltpu.SemaphoreType.DMA,
            pltpu.SemaphoreType.DMA,
            pltpu.SemaphoreType.DMA,
            pltpu.SemaphoreType.DMA,
            pltpu.SemaphoreType.DMA,
            pltpu.SemaphoreType.DMA,
            pltpu.SemaphoreType.DMA,
            pltpu.SemaphoreType.DMA,
        ],
    )
    def agg_kernel(t_ref, e_ref, z_ref, o_ref, iA, iB, r0, r1, r2, r3, acc,
                   siA, siB, sg0, sg1, sg2, sg3, ss0, ss1, ss2, ss3, sz):
        r = (r0, r1, r2, r3)
        sg = (sg0, sg1, sg2, sg3)
        ss = (ss0, ss1, ss2, ss3)
        cid = lax.axis_index("c")
        sid = lax.axis_index("s")
        wid = cid * _NT + sid
        zcopy = pltpu.async_copy(z_ref.at[pl.ds(sid * rows, rows)],
                                 acc.at[pl.ds(sid * rows, rows)], sz)
        pltpu.async_copy(e_ref.at[wid, 0], iA, siA)
        pltpu.async_copy(e_ref.at[wid, 1], iB, siB)
        pltpu.make_async_copy(e_ref.at[wid, 0], iA, siA).wait()
        for k in range(_BK):
            pltpu.async_copy(t_ref.at[iA.at[k, 0]], r[k], sg[k])
        zcopy.wait()
        plsc.subcore_barrier()

        @pl.loop(0, nblocks, step=2)
        def _(b):
            for k in range(_BK):
                pltpu.make_async_copy(t_ref.at[iA.at[k, 0]], r[k],
                                      sg[k]).wait()
                pltpu.async_copy(r[k], acc.at[iA.at[k, 1]], ss[k], add=True)
            pltpu.make_async_copy(e_ref.at[wid, b + 1], iB, siB).wait()
            for k in range(_BK):
                pltpu.make_async_copy(r[k], acc.at[iA.at[k, 1]],
                                      ss[k]).wait()
                pltpu.async_copy(t_ref.at[iB.at[k, 0]], r[k], sg[k])
            @pl.when(b + 2 < nblocks)
            def _():
                pltpu.async_copy(e_ref.at[wid, b + 2], iA, siA)

            for k in range(_BK):
                pltpu.make_async_copy(t_ref.at[iB.at[k, 0]], r[k],
                                      sg[k]).wait()
                pltpu.async_copy(r[k], acc.at[iB.at[k, 1]], ss[k], add=True)
            @pl.when(b + 2 < nblocks)
            def _():
                pltpu.make_async_copy(e_ref.at[wid, b + 2], iA, siA).wait()
            for k in range(_BK):
                pltpu.make_async_copy(r[k], acc.at[iB.at[k, 1]],
                                      ss[k]).wait()

                @pl.when(b + 2 < nblocks)
                def _():
                    pltpu.async_copy(t_ref.at[iA.at[k, 0]], r[k], sg[k])

            @pl.when(b + 3 < nblocks)
            def _():
                pltpu.async_copy(e_ref.at[wid, b + 3], iB, siB)

        plsc.subcore_barrier()
        pltpu.sync_copy(acc.at[pl.ds(sid * rows, rows)],
                        o_ref.at[cid, pl.ds(sid * rows, rows)])

    return agg_kernel


def _tc_scale0(x, counts, w):
    def body(x_ref, c_ref, w_ref, o_ref):
        cnt = c_ref[0] + c_ref[1]
        deg = jnp.maximum(cnt[:, 0:1] - cnt[:, 2:3] + 1.0, 1.0)
        nsrc = lax.rsqrt(deg)
        u = jnp.dot(nsrc[0:_N] * x_ref[...], w_ref[...],
                    preferred_element_type=jnp.float32)
        o_ref[0:_N, :] = u
        o_ref[_N:_NP, :] = jnp.zeros((_NP - _N, _D), jnp.float32)

    return pl.pallas_call(
        body, out_shape=jax.ShapeDtypeStruct((_NP, _D), jnp.float32),
    )(x, counts, w)


def _tc_epilogue(acc, hws, counts, b, g, be, wn):
    has_next = wn is not None
    outs = [jax.ShapeDtypeStruct((_N, _D), jnp.float32)]
    if has_next:
        outs.append(jax.ShapeDtypeStruct((_NP, _D), jnp.float32))

    def body(acc_ref, hws_ref, c_ref, b_ref, g_ref, be_ref, *rest):
        if has_next:
            wn_ref, h_ref, hn_ref = rest
        else:
            (h_ref,) = rest
        cnt = c_ref[0] + c_ref[1]
        c = cnt[0:_N, 2:3]
        ndst = lax.rsqrt(jnp.maximum(cnt[0:_N, 1:2] - c + 1.0, 1.0))
        agg = (acc_ref[0, 0:_N, :] + acc_ref[1, 0:_N, :]
               + (1.0 - c) * hws_ref[0:_N, :])
        pre = ndst * agg + b_ref[...]
        m = jnp.mean(pre, axis=0)
        msq = jnp.mean(pre * pre, axis=0)
        var = msq - m * m
        h = jnp.maximum(
            (pre - m) * lax.rsqrt(var + 1e-5) * g_ref[...] + be_ref[...], 0.0)
        h_ref[...] = h
        if has_next:
            nsrc = lax.rsqrt(jnp.maximum(cnt[0:_N, 0:1] - c + 1.0, 1.0))
            u = jnp.dot(nsrc * h, wn_ref[...],
                        preferred_element_type=jnp.float32)
            hn_ref[0:_N, :] = u
            hn_ref[_N:_NP, :] = jnp.zeros((_NP - _N, _D), jnp.float32)

    args = [acc, hws, counts, b, g, be] + ([wn] if has_next else [])
    res = pl.pallas_call(body, out_shape=outs)(*args)
    return tuple(res)


def kernel(x, edge_index, W0, b0, gamma0, beta0, W1, b1, gamma1, beta1,
           W2, b2, gamma2, beta2):
    e = edge_index.astype(jnp.int32)
    E = e.shape[1]
    block = _NW * _BK * _CH * 2
    epad = -(-E // block) * block
    nblocks = epad // (_NW * _BK * _CH)
    n_pad = epad - E
    dump = _NP - _N
    pidx = jnp.arange(n_pad, dtype=jnp.int32)
    psrc = _N + pidx % dump
    pdst = _N + (pidx * 7 + 13) % dump
    src = jnp.concatenate([e[0], psrc])
    dst = jnp.concatenate([e[1], pdst])
    e_all = jnp.stack([src, dst]).reshape(2, _NW, nblocks, _BK, _CH)
    e_all = jnp.transpose(e_all, (1, 2, 3, 0, 4))
    zc = jnp.zeros((_NT, _NP * 4 // _NT), jnp.float32)
    za = jnp.zeros((_NP, _D), jnp.float32)

    counts = _make_sc_counts(nblocks)(e_all, zc)
    counts = counts.reshape(_NC, _NP, 4)
    hws0 = _tc_scale0(x, counts, W0)

    acc0 = _make_sc_agg(nblocks)(hws0, e_all, za)
    h1, hws1 = _tc_epilogue(acc0, hws0, counts, b0, gamma0, beta0, W1)
    acc1 = _make_sc_agg(nblocks)(hws1, e_all, za)
    h2, hws2 = _tc_epilogue(acc1, hws1, counts, b1, gamma1, beta1, W2)
    acc2 = _make_sc_agg(nblocks)(hws2, e_all, za)
    (h3,) = _tc_epilogue(acc2, hws2, counts, b2, gamma2, beta2, None)

    return (x, h1, h2, h3)
